# baseline (speedup 1.0000x reference)
import jax
import jax.numpy as jnp
from jax.experimental import pallas as pl
from jax.experimental.pallas import tpu as pltpu

ATT_W = 1.0
REP_W = 1.0
BPOS_W = 10.0
BNEG_SIG_W = 3.0
BNEG_BG_W = 6.0
MARGIN_W = 10.0
THR = 0.5
MARGIN = 0.3
K = 64
T = 256


def _loss_kernel(beta_ref, sid_ref, cp_ref, cpt_ref, emb_ref, acc_ref):
    B = beta_ref.shape[0]
    N = beta_ref.shape[2]
    dn_t = (((1,), (1,)), ((), ()))
    ids = jax.lax.broadcasted_iota(jnp.int32, (K, 1), 0).astype(jnp.float32)
    nidx = jax.lax.broadcasted_iota(jnp.int32, (K, N), 1).astype(jnp.float32)

    total = 0.0
    cnt = 0.0
    pb = 0.0
    nb = 0.0
    pm = 0.0
    nm = 0.0
    for b in range(B):
        beta_b = beta_ref[b]
        sid = sid_ref[b]
        cp = cp_ref[b]
        emb = emb_ref[b]

        valid = (sid >= 0.0).astype(jnp.float32)
        cp_valid = cp * valid
        n_valid = jnp.sum(valid)
        n_cpv = jnp.sum(cp_valid)
        processed = jnp.where((n_valid > 0.0) & (n_cpv > 0.0), 1.0, 0.0)

        p = jax.nn.sigmoid(beta_b)
        log1pexp = jnp.log1p(jnp.exp(-jnp.abs(beta_b)))
        relu_b = jnp.maximum(beta_b, 0.0)
        ce1 = relu_b - beta_b + log1pexp
        bce0 = relu_b + log1pexp
        focal1 = 0.75 * (1.0 - p) * (1.0 - p) * ce1

        onehot = jnp.where(sid == ids, 1.0, 0.0)
        a_cp = onehot * cp
        cnt_cp = jnp.sum(a_cp, axis=1, keepdims=True)
        inst_size = jnp.sum(onehot, axis=1, keepdims=True)
        use = jnp.where(cnt_cp > 0.0, 1.0, 0.0)

        inst_focal = jnp.sum(a_cp * focal1, axis=1, keepdims=True)
        inst_mean = inst_focal / jnp.maximum(cnt_cp, 1.0)
        pos_accum = jnp.sum(use * inst_size * inst_mean)
        total_w = jnp.sum(use * inst_size)
        pos_bce_b = pos_accum / jnp.maximum(total_w, 1.0)

        non_cp = 1.0 - cp
        ncp_cnt = jnp.sum(non_cp)
        neg_bce_b = jnp.sum(bce0 * non_cp) / jnp.maximum(ncp_cnt, 1.0)

        cp_cnt = jnp.sum(cp)
        pos_margin_b = jnp.sum(jnp.maximum(THR + MARGIN - p, 0.0) * cp) / jnp.maximum(cp_cnt, 1.0)
        neg_margin_b = jnp.sum(jnp.maximum(p - (THR - MARGIN), 0.0) * non_cp) / jnp.maximum(ncp_cnt, 1.0)

        bg = jnp.where(sid == -1.0, 1.0, 0.0)
        bg_cnt = jnp.sum(bg)
        bg_bce = jnp.sum(bce0 * bg) / jnp.maximum(bg_cnt, 1.0)

        beta_loss = (BPOS_W * pos_bce_b + BNEG_SIG_W * neg_bce_b + BNEG_BG_W * bg_bce
                     + MARGIN_W * (pos_margin_b + neg_margin_b))

        embsq = emb * emb
        e2_col = jnp.sum(embsq, axis=1, keepdims=True)
        first = jnp.min(jnp.where(a_cp > 0.0, nidx, float(N)), axis=1, keepdims=True)
        sel = jnp.where(nidx == first, 1.0, 0.0)
        lhs = jnp.concatenate([onehot, sel], axis=0)
        rhs = jnp.concatenate([emb, e2_col], axis=1)
        s12c = jax.lax.dot_general(lhs, rhs, (((1,), (0,)), ((), ())),
                                   preferred_element_type=jnp.float32)
        D_ = emb.shape[1]
        s1 = s12c[:K, :D_]
        s2 = s12c[:K, D_:]
        c = s12c[K:, :D_]
        c2 = jnp.sum(c * c, axis=1, keepdims=True)
        cdots1 = jnp.sum(c * s1, axis=1, keepdims=True)
        att_sum = s2 - 2.0 * cdots1 + inst_size * c2
        att_mean = att_sum / jnp.maximum(inst_size, 1.0)
        attraction = ATT_W * jnp.sum(use * att_mean)

        nt = N // T
        BIG = 30000.0
        LOG2E = 1.4426950408889634
        cpv_col = cpt_ref[b]
        a_col = e2_col + BIG * (1.0 - cpv_col)
        ones_col = jnp.ones((N, 1), jnp.float32)
        rows_aug = jnp.concatenate([(2.0 * LOG2E) * emb, LOG2E * a_col,
                                    LOG2E * ones_col], axis=1)
        cols_aug = jnp.concatenate([emb, 0.0 - ones_col, 0.0 - a_col], axis=1)
        rep_sum = 0.0
        for ti in range(nt):
            rows = rows_aug[ti * T:(ti + 1) * T, :]
            for tj in range(ti, nt):
                cols = cols_aug[tj * T:(tj + 1) * T, :]
                g = jax.lax.dot_general(rows, cols, dn_t,
                                        preferred_element_type=jnp.float32)
                s = jnp.sum(jnp.exp2(g))
                rep_sum = rep_sum + (1.0 if ti == tj else 2.0) * s

        rep_mean = rep_sum / jnp.where(n_cpv > 1.0, n_cpv * n_cpv, 1.0)
        repulsion = jnp.where(n_cpv > 1.0, REP_W * rep_mean, 0.0)

        batch_loss = beta_loss + attraction + repulsion

        total = total + processed * batch_loss
        cnt = cnt + processed
        pb = processed * pos_bce_b + (1.0 - processed) * pb
        nb = processed * neg_bce_b + (1.0 - processed) * nb
        pm = processed * pos_margin_b + (1.0 - processed) * pm
        nm = processed * neg_margin_b + (1.0 - processed) * nm

    acc_ref[0] = jnp.where(cnt > 0.0, total / jnp.maximum(cnt, 1.0), 0.0)
    acc_ref[1] = pb
    acc_ref[2] = nb
    acc_ref[3] = pm
    acc_ref[4] = nm


def kernel(beta, embed, slice_id, is_cp):
    B, N, D = embed.shape
    beta2 = beta[..., 0].astype(jnp.float32).reshape(B, 1, N)
    sidf = slice_id.astype(jnp.float32).reshape(B, 1, N)
    cpf = is_cp.astype(jnp.float32).reshape(B, 1, N)
    cpvt = (is_cp & (slice_id >= 0)).astype(jnp.float32).reshape(B, N, 1)

    acc = pl.pallas_call(
        _loss_kernel,
        out_specs=pl.BlockSpec(memory_space=pltpu.MemorySpace.SMEM),
        out_shape=jax.ShapeDtypeStruct((8,), jnp.float32),
    )(beta2, sidf, cpf, cpvt, embed)

    return (acc[0], acc[1], acc[2], acc[3], acc[4])


# speedup vs baseline: 1.0769x; 1.0769x over previous
import jax
import jax.numpy as jnp
from jax.experimental import pallas as pl
from jax.experimental.pallas import tpu as pltpu

ATT_W = 1.0
REP_W = 1.0
BPOS_W = 10.0
BNEG_SIG_W = 3.0
BNEG_BG_W = 6.0
MARGIN_W = 10.0
THR = 0.5
MARGIN = 0.3
K = 64
T = 256


def _loss_kernel(beta_ref, sid_ref, cp_ref, cpt_ref, emb_ref, acc_ref):
    B = beta_ref.shape[0]
    N = beta_ref.shape[2]
    dn_t = (((1,), (1,)), ((), ()))
    ids = jax.lax.broadcasted_iota(jnp.int32, (K, 1), 0).astype(jnp.float32)
    nidx = jax.lax.broadcasted_iota(jnp.int32, (K, N), 1).astype(jnp.float32)

    total = 0.0
    cnt = 0.0
    pb = 0.0
    nb = 0.0
    pm = 0.0
    nm = 0.0
    for b in range(B):
        beta_b = beta_ref[b]
        sid = sid_ref[b]
        cp = cp_ref[b]
        emb = emb_ref[b]

        valid = (sid >= 0.0).astype(jnp.float32)
        cp_valid = cp * valid
        n_valid = jnp.sum(valid)
        n_cpv = jnp.sum(cp_valid)
        processed = jnp.where((n_valid > 0.0) & (n_cpv > 0.0), 1.0, 0.0)

        p = jax.nn.sigmoid(beta_b)
        log1pexp = jnp.log1p(jnp.exp(-jnp.abs(beta_b)))
        relu_b = jnp.maximum(beta_b, 0.0)
        ce1 = relu_b - beta_b + log1pexp
        bce0 = relu_b + log1pexp
        focal1 = 0.75 * (1.0 - p) * (1.0 - p) * ce1

        onehot = jnp.where(sid == ids, 1.0, 0.0)
        a_cp = onehot * cp
        cnt_cp = jnp.sum(a_cp, axis=1, keepdims=True)
        inst_size = jnp.sum(onehot, axis=1, keepdims=True)
        use = jnp.where(cnt_cp > 0.0, 1.0, 0.0)

        inst_focal = jnp.sum(a_cp * focal1, axis=1, keepdims=True)
        inst_mean = inst_focal / jnp.maximum(cnt_cp, 1.0)
        pos_accum = jnp.sum(use * inst_size * inst_mean)
        total_w = jnp.sum(use * inst_size)
        pos_bce_b = pos_accum / jnp.maximum(total_w, 1.0)

        non_cp = 1.0 - cp
        ncp_cnt = jnp.sum(non_cp)
        neg_bce_b = jnp.sum(bce0 * non_cp) / jnp.maximum(ncp_cnt, 1.0)

        cp_cnt = jnp.sum(cp)
        pos_margin_b = jnp.sum(jnp.maximum(THR + MARGIN - p, 0.0) * cp) / jnp.maximum(cp_cnt, 1.0)
        neg_margin_b = jnp.sum(jnp.maximum(p - (THR - MARGIN), 0.0) * non_cp) / jnp.maximum(ncp_cnt, 1.0)

        bg = jnp.where(sid == -1.0, 1.0, 0.0)
        bg_cnt = jnp.sum(bg)
        bg_bce = jnp.sum(bce0 * bg) / jnp.maximum(bg_cnt, 1.0)

        beta_loss = (BPOS_W * pos_bce_b + BNEG_SIG_W * neg_bce_b + BNEG_BG_W * bg_bce
                     + MARGIN_W * (pos_margin_b + neg_margin_b))

        embsq = emb * emb
        ones_d2 = jnp.ones((1, emb.shape[1]), jnp.float32)
        e2_col = jax.lax.dot_general(embsq, ones_d2, dn_t,
                                     preferred_element_type=jnp.float32)
        s1 = jax.lax.dot_general(onehot, emb, (((1,), (0,)), ((), ())),
                                 preferred_element_type=jnp.float32)
        s2 = jax.lax.dot_general(onehot, e2_col, (((1,), (0,)), ((), ())),
                                 preferred_element_type=jnp.float32)
        first = jnp.min(jnp.where(a_cp > 0.0, nidx, float(N)), axis=1, keepdims=True)
        sel = jnp.where(nidx == first, 1.0, 0.0)
        c = jax.lax.dot_general(sel, emb, (((1,), (0,)), ((), ())),
                                preferred_element_type=jnp.float32)
        c2 = jnp.sum(c * c, axis=1, keepdims=True)
        cdots1 = jnp.sum(c * s1, axis=1, keepdims=True)
        att_sum = s2 - 2.0 * cdots1 + inst_size * c2
        att_mean = att_sum / jnp.maximum(inst_size, 1.0)
        attraction = ATT_W * jnp.sum(use * att_mean)

        nt = N // T
        BIG = 30000.0
        LOG2E = 1.4426950408889634
        cpv_col = cpt_ref[b]
        a_col = e2_col + BIG * (1.0 - cpv_col)
        ones_col = jnp.ones((N, 1), jnp.float32)
        rows_aug = jnp.concatenate([(2.0 * LOG2E) * emb, LOG2E * a_col,
                                    LOG2E * ones_col], axis=1)
        cols_aug = jnp.concatenate([emb, 0.0 - ones_col, 0.0 - a_col], axis=1)
        rep_sum = 0.0
        for ti in range(nt):
            rows = rows_aug[ti * T:(ti + 1) * T, :]
            for tj in range(ti, nt):
                cols = cols_aug[tj * T:(tj + 1) * T, :]
                g = jax.lax.dot_general(rows, cols, dn_t,
                                        preferred_element_type=jnp.float32)
                s = jnp.sum(jnp.exp2(g))
                rep_sum = rep_sum + (1.0 if ti == tj else 2.0) * s

        rep_mean = rep_sum / jnp.where(n_cpv > 1.0, n_cpv * n_cpv, 1.0)
        repulsion = jnp.where(n_cpv > 1.0, REP_W * rep_mean, 0.0)

        batch_loss = beta_loss + attraction + repulsion

        total = total + processed * batch_loss
        cnt = cnt + processed
        pb = processed * pos_bce_b + (1.0 - processed) * pb
        nb = processed * neg_bce_b + (1.0 - processed) * nb
        pm = processed * pos_margin_b + (1.0 - processed) * pm
        nm = processed * neg_margin_b + (1.0 - processed) * nm

    acc_ref[0] = jnp.where(cnt > 0.0, total / jnp.maximum(cnt, 1.0), 0.0)
    acc_ref[1] = pb
    acc_ref[2] = nb
    acc_ref[3] = pm
    acc_ref[4] = nm


def kernel(beta, embed, slice_id, is_cp):
    B, N, D = embed.shape
    beta2 = beta[..., 0].astype(jnp.float32).reshape(B, 1, N)
    sidf = slice_id.astype(jnp.float32).reshape(B, 1, N)
    cpf = is_cp.astype(jnp.float32).reshape(B, 1, N)
    cpvt = (is_cp & (slice_id >= 0)).astype(jnp.float32).reshape(B, N, 1)

    acc = pl.pallas_call(
        _loss_kernel,
        out_specs=pl.BlockSpec(memory_space=pltpu.MemorySpace.SMEM),
        out_shape=jax.ShapeDtypeStruct((8,), jnp.float32),
    )(beta2, sidf, cpf, cpvt, embed)

    return (acc[0], acc[1], acc[2], acc[3], acc[4])
